# baseline (speedup 1.0000x reference)
import functools

import numpy as np

import jax
import jax.numpy as jnp
from jax import lax
from jax.experimental import pallas as pl
from jax.experimental.pallas import tpu as pltpu
from jax.experimental.pallas import tpu_sc as plsc

_FEATURE_DIMS = [119, 5, 12, 12, 10, 6, 6, 2, 2]
_NF = len(_FEATURE_DIMS)
_C = 256
_K = sum(_FEATURE_DIMS)
_K_PAD = 176
_STRIDE = 257
_NB = 32
_NW = 32
_NSB = 100000 // _NB
_REM = _NSB - (_NSB // _NW) * _NW


def _sc_body(table_hbm, idx_hbm, out_hbm, table_v, idx_v, out_v):
    wid = lax.axis_index("s") * 2 + lax.axis_index("c")
    pltpu.sync_copy(table_hbm, table_v)
    n_sb = jnp.where(wid < _REM, _NSB // _NW + 1, _NSB // _NW)

    def sb_body(k, carry):
        sb = wid + k * _NW
        iota = lax.broadcasted_iota(jnp.int32, (16,), 0)
        pltpu.sync_copy(idx_hbm.at[sb], idx_v)
        for g in range(_NB // 16):
            rows = [idx_v[i, pl.ds(g * 16, 16)] for i in range(_NF)]
            base = [r * _STRIDE for r in rows]
            dst_rows = iota + (g * 16)

            @plsc.parallel_loop(0, _C, 1, unroll=8)
            def ch_body(ch, base=base, dst_rows=dst_rows, iota=iota):
                colv = (ch & -16) + ((iota + ch) & 15)
                gs = [plsc.load_gather(table_v, [base[i] + colv])
                      for i in range(_NF)]
                while len(gs) > 1:
                    gs = [a + b for a, b in zip(gs[::2], gs[1::2])] + (
                        [gs[-1]] if len(gs) % 2 else [])
                plsc.store_scatter(out_v, [dst_rows, colv], gs[0])
        pltpu.sync_copy(out_v, out_hbm.at[pl.ds(sb * _NB, _NB), :])
        return carry

    lax.fori_loop(0, n_sb, sb_body, 0)


def kernel(x, batch, emb_0, emb_1, emb_2, emb_3, emb_4, emb_5, emb_6, emb_7,
           emb_8):
    del batch
    embs = [emb_0, emb_1, emb_2, emb_3, emb_4, emb_5, emb_6, emb_7, emb_8]
    table = jnp.concatenate(embs, axis=0)
    table = jnp.pad(table, ((0, _K_PAD - _K), (0, _STRIDE - _C))).reshape(-1)

    offs = np.zeros((1, _NF), np.int32)
    acc = 0
    for i, d in enumerate(_FEATURE_DIMS):
        offs[0, i] = acc
        acc += d
    n = x.shape[0]
    flat = x.astype(jnp.int32) + jnp.asarray(offs)
    idx = jnp.transpose(flat.reshape(_NSB, _NB, _NF), (0, 2, 1))
    idx = jnp.concatenate(
        [idx, jnp.zeros((_NSB, 16 - _NF, _NB), jnp.int32)], axis=1)

    mesh = plsc.VectorSubcoreMesh(
        core_axis_name="c", subcore_axis_name="s", num_cores=2,
        num_subcores=16)
    k = pl.kernel(
        _sc_body,
        out_type=jax.ShapeDtypeStruct((n, _C), jnp.float32),
        mesh=mesh,
        scratch_types=[
            pltpu.VMEM((_K_PAD * _STRIDE,), jnp.float32),
            pltpu.VMEM((16, _NB), jnp.int32),
            pltpu.VMEM((_NB, _C), jnp.float32),
        ],
        compiler_params=pltpu.CompilerParams(needs_layout_passes=False),
    )
    return k(table, idx)


# speedup vs baseline: 1.3870x; 1.3870x over previous
import functools

import numpy as np

import jax
import jax.numpy as jnp
from jax import lax
from jax.experimental import pallas as pl
from jax.experimental.pallas import tpu as pltpu
from jax.experimental.pallas import tpu_sc as plsc

_FEATURE_DIMS = [119, 5, 12, 12, 10, 6, 6, 2, 2]
_NF = len(_FEATURE_DIMS)
_C = 256
_K = sum(_FEATURE_DIMS)
_K_PAD = 176
_STRIDE = 257
_NB = 32
_NW = 32
_NSB = 100000 // _NB
_REM = _NSB - (_NSB // _NW) * _NW


def _sc_body(table_hbm, idx_hbm, out_hbm, table_v, idx_v, out_v):
    wid = lax.axis_index("s") * 2 + lax.axis_index("c")
    pltpu.sync_copy(table_hbm, table_v)
    n_sb = jnp.where(wid < _REM, _NSB // _NW + 1, _NSB // _NW)

    def sb_body(k, carry):
        sb = wid + k * _NW
        iota = lax.broadcasted_iota(jnp.int32, (16,), 0)
        pltpu.sync_copy(idx_hbm.at[sb], idx_v)
        for g in range(_NB // 16):
            rows = [idx_v[i, pl.ds(g * 16, 16)] for i in range(_NF)]
            base = [r * _STRIDE for r in rows]
            dst_rows = iota + (g * 16)

            @plsc.parallel_loop(0, _C, 1, unroll=4)
            def ch_body(ch, base=base, dst_rows=dst_rows, iota=iota):
                colv = (ch & -16) + ((iota + ch) & 15)
                gs = [plsc.load_gather(table_v, [base[i] + colv])
                      for i in range(_NF)]
                while len(gs) > 1:
                    gs = [a + b for a, b in zip(gs[::2], gs[1::2])] + (
                        [gs[-1]] if len(gs) % 2 else [])
                plsc.store_scatter(out_v, [dst_rows, colv], gs[0])
        pltpu.sync_copy(out_v, out_hbm.at[pl.ds(sb * _NB, _NB), :])
        return carry

    lax.fori_loop(0, n_sb, sb_body, 0)


def kernel(x, batch, emb_0, emb_1, emb_2, emb_3, emb_4, emb_5, emb_6, emb_7,
           emb_8):
    del batch
    embs = [emb_0, emb_1, emb_2, emb_3, emb_4, emb_5, emb_6, emb_7, emb_8]
    table = jnp.concatenate(embs, axis=0)
    table = jnp.pad(table, ((0, _K_PAD - _K), (0, _STRIDE - _C))).reshape(-1)

    offs = np.zeros((1, _NF), np.int32)
    acc = 0
    for i, d in enumerate(_FEATURE_DIMS):
        offs[0, i] = acc
        acc += d
    n = x.shape[0]
    flat = x.astype(jnp.int32) + jnp.asarray(offs)
    idx = jnp.transpose(flat.reshape(_NSB, _NB, _NF), (0, 2, 1))
    idx = jnp.concatenate(
        [idx, jnp.zeros((_NSB, 16 - _NF, _NB), jnp.int32)], axis=1)

    mesh = plsc.VectorSubcoreMesh(
        core_axis_name="c", subcore_axis_name="s", num_cores=2,
        num_subcores=16)
    k = pl.kernel(
        _sc_body,
        out_type=jax.ShapeDtypeStruct((n, _C), jnp.float32),
        mesh=mesh,
        scratch_types=[
            pltpu.VMEM((_K_PAD * _STRIDE,), jnp.float32),
            pltpu.VMEM((16, _NB), jnp.int32),
            pltpu.VMEM((_NB, _C), jnp.float32),
        ],
        compiler_params=pltpu.CompilerParams(needs_layout_passes=False),
    )
    return k(table, idx)
